# Initial kernel scaffold; baseline (speedup 1.0000x reference)
import jax
import jax.numpy as jnp
from jax.experimental import pallas as pl


def kernel(feat, edge_index, W_fc1, b_fc1, W_c1, b_c1, W_c2, b_c2, W_fc2, b_fc2):
    raise NotImplementedError("write your pallas kernel here")


# speedup vs baseline: 5.0346x; 5.0346x over previous
import functools
import math

import jax
import jax.numpy as jnp
from jax import lax
from jax.experimental import pallas as pl
from jax.experimental.pallas import tpu as pltpu
from jax.experimental.pallas import tpu_sc as plsc

N = 10000
E = 320000
D = 128
ALPHA = 0.2
BETA1 = math.log(2.0)
BETA2 = math.log(1.5)

NC = 2
NS = 16
NW = NC * NS
EPW = E // NW
NP = 10240
C = 80
CH = EPW // C
RPT = NP // NS

_mesh = plsc.VectorSubcoreMesh(core_axis_name="c", subcore_axis_name="s")


@functools.partial(
    pl.kernel,
    mesh=_mesh,
    out_type=jax.ShapeDtypeStruct((NC, NP, D), jnp.float32),
    scratch_types=[
        pltpu.VMEM((CH, C), jnp.int32),
        pltpu.VMEM((C, D), jnp.float32),
        pltpu.VMEM_SHARED((NP, D), jnp.float32),
    ],
)
def _sc_degree(idx_hbm, ones_hbm, zeros_hbm, out_hbm, idxv, ones_v, acc):
    cid = lax.axis_index("c")
    sid = lax.axis_index("s")
    wid = cid * NS + sid
    r0 = sid * RPT
    pltpu.sync_copy(zeros_hbm, acc.at[pl.ds(r0, RPT)])
    pltpu.sync_copy(idx_hbm.at[wid], idxv)
    pltpu.sync_copy(ones_hbm, ones_v)
    plsc.subcore_barrier()

    def body(j, carry):
        pltpu.sync_copy(ones_v, acc.at[idxv.at[j]], add=True)
        return carry

    lax.fori_loop(0, CH, body, 0)
    plsc.subcore_barrier()
    pltpu.sync_copy(acc.at[pl.ds(r0, RPT)], out_hbm.at[cid].at[pl.ds(r0, RPT)])


@functools.partial(
    pl.kernel,
    mesh=_mesh,
    out_type=jax.ShapeDtypeStruct((NC, NP, D), jnp.float32),
    scratch_types=[
        pltpu.VMEM((CH, C), jnp.int32),
        pltpu.VMEM((CH, C), jnp.int32),
        pltpu.VMEM((C, D), jnp.float32),
        pltpu.VMEM_SHARED((NP, D), jnp.float32),
        pltpu.SemaphoreType.DMA,
    ],
)
def _sc_segsum(h_hbm, src_hbm, dst_hbm, zeros_hbm, out_hbm,
               sidx, didx, rows, acc, sem):
    cid = lax.axis_index("c")
    sid = lax.axis_index("s")
    wid = cid * NS + sid
    r0 = sid * RPT
    pltpu.sync_copy(zeros_hbm, acc.at[pl.ds(r0, RPT)])
    pltpu.sync_copy(src_hbm.at[wid], sidx)
    pltpu.sync_copy(dst_hbm.at[wid], didx)
    plsc.subcore_barrier()

    def body(j, carry):
        pltpu.async_copy(h_hbm.at[sidx.at[j]], rows, sem).wait()
        pltpu.sync_copy(rows, acc.at[didx.at[j]], add=True)
        return carry

    lax.fori_loop(0, CH, body, 0)
    plsc.subcore_barrier()
    pltpu.sync_copy(acc.at[pl.ds(r0, RPT)], out_hbm.at[cid].at[pl.ds(r0, RPT)])


RB = 1000


def _scales(cnt_ref):
    deg = jnp.maximum(cnt_ref[0, :, 0] + cnt_ref[1, :, 0], 1.0)
    return lax.rsqrt(deg)[:, None]


def _tc1_body(feat_ref, w1_ref, b1_ref, cs_ref, x0_ref, h1_ref):
    x0 = lax.dot_general(feat_ref[...], w1_ref[...],
                         (((1,), (1,)), ((), ())),
                         preferred_element_type=jnp.float32)
    x0 = jnp.maximum(x0 + b1_ref[...], 0.0)
    x0_ref[...] = x0
    h1_ref[...] = x0 * _scales(cs_ref)


def _tc2_body(p_ref, x0_ref, cs_ref, cd_ref, w_ref, b_ref, h2_ref):
    agg = (p_ref[0] + p_ref[1]) * _scales(cd_ref)
    t = (1.0 - ALPHA) * agg + ALPHA * x0_ref[...]
    u = (1.0 - BETA1) * t + BETA1 * lax.dot_general(
        t, w_ref[...], (((1,), (0,)), ((), ())),
        preferred_element_type=jnp.float32)
    x1 = jnp.maximum(u + b_ref[...], 0.0)
    h2_ref[...] = x1 * _scales(cs_ref)


def _tc3_body(p_ref, x0_ref, cd_ref, w_ref, b_ref, w2_ref, b2_ref, out_ref):
    agg = (p_ref[0] + p_ref[1]) * _scales(cd_ref)
    t = (1.0 - ALPHA) * agg + ALPHA * x0_ref[...]
    u = (1.0 - BETA2) * t + BETA2 * lax.dot_general(
        t, w_ref[...], (((1,), (0,)), ((), ())),
        preferred_element_type=jnp.float32)
    u = u + b_ref[...]
    out = lax.dot_general(u, w2_ref[...], (((1,), (1,)), ((), ())),
                          preferred_element_type=jnp.float32)
    out_ref[...] = out + b2_ref[...]


_row_spec = pl.BlockSpec((RB, D), lambda i: (i, 0))
_mat_spec = pl.BlockSpec((D, D), lambda i: (0, 0))
_vec_spec = pl.BlockSpec((1, D), lambda i: (0, 0))
_cnt_spec = pl.BlockSpec((NC, RB, D), lambda i: (0, i, 0))
_par_spec = pl.BlockSpec((NC, RB, D), lambda i: (0, i, 0))
_GRID = (N // RB,)


def _tc1(feat, w1, b1, cs):
    return pl.pallas_call(
        _tc1_body,
        grid=_GRID,
        in_specs=[_row_spec, _mat_spec, _vec_spec, _cnt_spec],
        out_specs=[_row_spec, _row_spec],
        out_shape=[jax.ShapeDtypeStruct((N, D), jnp.float32)] * 2,
    )(feat, w1, b1, cs)


def _tc2(p, x0, cs, cd, w, b):
    return pl.pallas_call(
        _tc2_body,
        grid=_GRID,
        in_specs=[_par_spec, _row_spec, _cnt_spec, _cnt_spec, _mat_spec,
                  _vec_spec],
        out_specs=_row_spec,
        out_shape=jax.ShapeDtypeStruct((N, D), jnp.float32),
    )(p, x0, cs, cd, w, b)


def _tc3(p, x0, cd, w, b, w2, b2):
    return pl.pallas_call(
        _tc3_body,
        grid=_GRID,
        in_specs=[_par_spec, _row_spec, _cnt_spec, _mat_spec, _vec_spec,
                  _mat_spec, _vec_spec],
        out_specs=_row_spec,
        out_shape=jax.ShapeDtypeStruct((N, D), jnp.float32),
    )(p, x0, cd, w, b, w2, b2)


def kernel(feat, edge_index, W_fc1, b_fc1, W_c1, b_c1, W_c2, b_c2, W_fc2, b_fc2):
    src = edge_index[0].astype(jnp.int32).reshape(NW, CH, C)
    dst = edge_index[1].astype(jnp.int32).reshape(NW, CH, C)
    ones_r = jnp.ones((C, D), jnp.float32)
    zrows = jnp.zeros((RPT, D), jnp.float32)

    cs = _sc_degree(src, ones_r, zrows)
    cd = _sc_degree(dst, ones_r, zrows)
    x0, h1 = _tc1(feat, W_fc1, b_fc1.reshape(1, D), cs)
    p1 = _sc_segsum(h1, src, dst, zrows)
    h2 = _tc2(p1, x0, cs, cd, W_c1, b_c1.reshape(1, D))
    p2 = _sc_segsum(h2, src, dst, zrows)
    return _tc3(p2, x0, cd, W_c2, b_c2.reshape(1, D), W_fc2, b_fc2.reshape(1, D))
